# baseline (speedup 1.0000x reference)
import functools
import jax
import jax.numpy as jnp
from jax import lax
from jax.experimental import pallas as pl
from jax.experimental.pallas import tpu as pltpu
from jax.experimental.pallas import tpu_sc as plsc

_D = 1024
_H = 4096
_E = 8
_K = 2
_N = 2048
_B = 256
_HS = 2
_P = _N * _K
_G = _P // _B + _E
_ROWS = _G * _B

_INTERP = False


def _gate_kernel(x_ref, gw_ref, gb_ref, w_ref, i_ref):
    logits = jnp.dot(x_ref[...], gw_ref[...],
                     preferred_element_type=jnp.float32) + gb_ref[...]
    cols = jax.lax.broadcasted_iota(jnp.int32, logits.shape, 1)
    m1 = jnp.max(logits, axis=1)
    i1 = jnp.argmax(logits, axis=1).astype(jnp.int32)
    masked = jnp.where(cols == i1[:, None], -jnp.inf, logits)
    m2 = jnp.max(masked, axis=1)
    i2 = jnp.argmax(masked, axis=1).astype(jnp.int32)
    e2 = jnp.exp(m2 - m1)
    w1 = 1.0 / (1.0 + e2)
    w2 = e2 / (1.0 + e2)
    w_ref[...] = jnp.stack([w1, w2], axis=1)
    i_ref[...] = jnp.stack([i1, i2], axis=1)


def _gate(x, gate_w, gate_b):
    bt = 256
    return pl.pallas_call(
        _gate_kernel,
        grid=(_N // bt,),
        in_specs=[
            pl.BlockSpec((bt, _D), lambda t: (t, 0)),
            pl.BlockSpec((_D, _E), lambda t: (0, 0)),
            pl.BlockSpec((_E,), lambda t: (0,)),
        ],
        out_specs=[
            pl.BlockSpec((bt, _K), lambda t: (t, 0)),
            pl.BlockSpec((bt, _K), lambda t: (t, 0)),
        ],
        out_shape=[
            jax.ShapeDtypeStruct((_N, _K), jnp.float32),
            jax.ShapeDtypeStruct((_N, _K), jnp.int32),
        ],
        interpret=_INTERP,
    )(x, gate_w, gate_b)


def _route(idx, w):
    idxf = idx.reshape(-1)
    onehot = (idxf[:, None] == jnp.arange(_E)[None, :]).astype(jnp.int32)
    counts = onehot.sum(0)
    pc = ((counts + _B - 1) // _B) * _B
    ends = jnp.cumsum(pc)
    off = ends - pc
    ranks = jnp.cumsum(onehot, 0) - onehot
    r = (ranks * onehot).sum(1)
    pos = off[idxf] + r
    src = jnp.zeros((_ROWS,), jnp.int32).at[pos].set(
        jnp.arange(_P, dtype=jnp.int32) // _K)
    ws = jnp.zeros((_ROWS,), jnp.float32).at[pos].set(w.reshape(-1))
    n_used = jnp.sum(pc) // _B
    te_raw = jnp.searchsorted(ends, jnp.arange(_G, dtype=jnp.int32) * _B,
                              side='right').astype(jnp.int32)
    te = jnp.minimum(te_raw, te_raw[n_used - 1])
    return te, n_used.reshape(1).astype(jnp.int32), src, ws, pos.reshape(_N, _K)


def _gelu(a):
    return a * 0.5 * (1.0 + jax.lax.erf(a * 0.7071067811865476))


def _ffn_kernel(te_ref, nu_ref, xs_ref, w1_ref, b1_ref, w2_ref, b2_ref,
                ws_ref, out_ref):
    hs = pl.program_id(0)
    g = pl.program_id(1)

    @pl.when(g < nu_ref[0])
    def _():
        xb = xs_ref[...].astype(jnp.bfloat16)
        a = jnp.dot(xb, w1_ref[0].astype(jnp.bfloat16),
                    preferred_element_type=jnp.float32) + b1_ref[0]
        y = jnp.dot(_gelu(a).astype(jnp.bfloat16),
                    w2_ref[0].astype(jnp.bfloat16),
                    preferred_element_type=jnp.float32)
        scale = jnp.where(hs == 0, 1.0, 0.0)
        out_ref[...] = (y + scale * b2_ref[0]) * ws_ref[...]


def _ffn(te, n_used, xs, W1, b1, W2, b2, ws):
    hh = _H // _HS
    grid_spec = pltpu.PrefetchScalarGridSpec(
        num_scalar_prefetch=2,
        grid=(_HS, _G),
        in_specs=[
            pl.BlockSpec((_B, _D), lambda hs, g, te, nu: (g, 0)),
            pl.BlockSpec((1, _D, hh), lambda hs, g, te, nu: (te[g], 0, hs)),
            pl.BlockSpec((1, 1, hh), lambda hs, g, te, nu: (te[g], 0, hs)),
            pl.BlockSpec((1, hh, _D), lambda hs, g, te, nu: (te[g], hs, 0)),
            pl.BlockSpec((1, 1, _D), lambda hs, g, te, nu: (te[g], 0, 0)),
            pl.BlockSpec((_B, 1), lambda hs, g, te, nu: (g, 0)),
        ],
        out_specs=pl.BlockSpec((_B, _D), lambda hs, g, te, nu: (hs * _G + g, 0)),
    )
    return pl.pallas_call(
        _ffn_kernel,
        grid_spec=grid_spec,
        out_shape=jax.ShapeDtypeStruct((_HS * _ROWS, _D), jnp.float32),
        interpret=_INTERP,
    )(te, n_used, xs, W1, b1.reshape(_E, 1, _H), W2,
      b2.reshape(_E, 1, _D), ws.reshape(_ROWS, 1))


_NW = 32
_RPW = _ROWS // _NW
_GCH = 32
_GNB = 3
_TPW = _N // _NW
_TCH = 8
_TNB = 2


def _sc_mesh():
    return plsc.VectorSubcoreMesh(core_axis_name="c", subcore_axis_name="s")


def _gather_sc(x, src):
    @functools.partial(
        pl.kernel, mesh=_sc_mesh(),
        out_type=jax.ShapeDtypeStruct((_ROWS, _D), jnp.float32),
        scratch_types=[
            pltpu.VMEM((_RPW // _GCH, _GCH), jnp.int32),
            pltpu.VMEM((_GNB, _GCH, _D), jnp.float32),
        ] + [pltpu.SemaphoreType.DMA] * (2 * _GNB),
    )
    def k(x_hbm, src_hbm, xs_hbm, idx_v, rows_v, *sems):
        gsems = sems[:_GNB]
        wsems = sems[_GNB:]
        wid = lax.axis_index("s") * 2 + lax.axis_index("c")
        base = wid * _RPW
        ncv = _RPW // _GCH
        for c in range(ncv):
            pltpu.sync_copy(src_hbm.at[pl.ds(base + c * _GCH, _GCH)],
                            idx_v.at[c])
        gh = [None] * ncv
        wh = [None] * ncv

        def gather(c):
            b = c % _GNB
            return pltpu.async_copy(
                x_hbm.at[idx_v.at[c]], rows_v.at[b], gsems[b])

        def write(c):
            b = c % _GNB
            return pltpu.async_copy(
                rows_v.at[b], xs_hbm.at[pl.ds(base + c * _GCH, _GCH)],
                wsems[b])

        for c in range(ncv):
            if c >= _GNB:
                wh[c - _GNB].wait()
            gh[c] = gather(c)
            if c >= 1:
                gh[c - 1].wait()
                wh[c - 1] = write(c - 1)
        gh[ncv - 1].wait()
        wh[ncv - 1] = write(ncv - 1)
        for c in range(max(0, ncv - _GNB), ncv):
            wh[c].wait()

    return k(x, src)


def _combine_sc(ys, ps):
    @functools.partial(
        pl.kernel, mesh=_sc_mesh(),
        out_type=jax.ShapeDtypeStruct((_N, _D), jnp.float32),
        scratch_types=[
            pltpu.VMEM((4, _TPW), jnp.int32),
            pltpu.VMEM((_TNB, 4, _TCH, _D), jnp.float32),
        ] + [pltpu.SemaphoreType.DMA] * (5 * _TNB),
    )
    def k(ys_hbm, ps_hbm, out_hbm, ix_v, buf_v, *sems):
        gsems = sems[:4 * _TNB]
        wsems = sems[4 * _TNB:]
        wid = lax.axis_index("s") * 2 + lax.axis_index("c")
        base = wid * _TPW
        for q in range(4):
            pltpu.sync_copy(ps_hbm.at[q, pl.ds(base, _TPW)], ix_v.at[q])
        ncv = _TPW // _TCH
        gh = [None] * ncv
        wh = [None] * ncv

        def gather(c):
            b = c % _TNB
            isl = pl.ds(c * _TCH, _TCH)
            return [
                pltpu.async_copy(ys_hbm.at[ix_v.at[q, isl]],
                                 buf_v.at[b, q], gsems[4 * b + q])
                for q in range(4)
            ]

        def accum_write(c):
            b = c % _TNB

            def body(r, carry):
                for j in range(_D // 16):
                    sl = pl.ds(j * 16, 16)
                    buf_v[b, 0, r, sl] = (
                        (buf_v[b, 0, r, sl] + buf_v[b, 1, r, sl])
                        + (buf_v[b, 2, r, sl] + buf_v[b, 3, r, sl]))
                return carry

            lax.fori_loop(0, _TCH, body, 0)
            return pltpu.async_copy(
                buf_v.at[b, 0], out_hbm.at[pl.ds(base + c * _TCH, _TCH)],
                wsems[b])

        for c in range(ncv):
            if c >= _TNB:
                wh[c - _TNB].wait()
            gh[c] = gather(c)
            if c >= 1:
                for cp in gh[c - 1]:
                    cp.wait()
                wh[c - 1] = accum_write(c - 1)
        for cp in gh[ncv - 1]:
            cp.wait()
        wh[ncv - 1] = accum_write(ncv - 1)
        for c in range(max(0, ncv - _TNB), ncv):
            wh[c].wait()

    return k(ys, ps)


def kernel(x, gate_w, gate_b, W1, b1, W2, b2):
    w, idx = _gate(x, gate_w, gate_b)
    te, n_used, src, ws, pos = _route(idx, w)
    xs = _gather_sc(x, src)
    ys = _ffn(te, n_used, xs, W1, b1, W2, b2, ws)
    ps = jnp.stack([pos[:, 0], pos[:, 1],
                    pos[:, 0] + _ROWS, pos[:, 1] + _ROWS])
    out = _combine_sc(ys, ps)
    return out


# speedup vs baseline: 1.1817x; 1.1817x over previous
import functools
import jax
import jax.numpy as jnp
from jax import lax
from jax.experimental import pallas as pl
from jax.experimental.pallas import tpu as pltpu
from jax.experimental.pallas import tpu_sc as plsc

_D = 1024
_H = 4096
_E = 8
_K = 2
_N = 2048
_B = 256
_HS = 2
_P = _N * _K
_G = _P // _B + _E
_ROWS = _G * _B

_INTERP = False


def _gate_kernel(x_ref, gw_ref, gb_ref, w_ref, i_ref):
    logits = jnp.dot(x_ref[...], gw_ref[...],
                     preferred_element_type=jnp.float32) + gb_ref[...]
    cols = jax.lax.broadcasted_iota(jnp.int32, logits.shape, 1)
    m1 = jnp.max(logits, axis=1)
    i1 = jnp.argmax(logits, axis=1).astype(jnp.int32)
    masked = jnp.where(cols == i1[:, None], -jnp.inf, logits)
    m2 = jnp.max(masked, axis=1)
    i2 = jnp.argmax(masked, axis=1).astype(jnp.int32)
    e2 = jnp.exp(m2 - m1)
    w1 = 1.0 / (1.0 + e2)
    w2 = e2 / (1.0 + e2)
    w_ref[...] = jnp.stack([w1, w2], axis=1)
    i_ref[...] = jnp.stack([i1, i2], axis=1)


def _gate(x, gate_w, gate_b):
    bt = 256
    return pl.pallas_call(
        _gate_kernel,
        grid=(_N // bt,),
        in_specs=[
            pl.BlockSpec((bt, _D), lambda t: (t, 0)),
            pl.BlockSpec((_D, _E), lambda t: (0, 0)),
            pl.BlockSpec((_E,), lambda t: (0,)),
        ],
        out_specs=[
            pl.BlockSpec((bt, _K), lambda t: (t, 0)),
            pl.BlockSpec((bt, _K), lambda t: (t, 0)),
        ],
        out_shape=[
            jax.ShapeDtypeStruct((_N, _K), jnp.float32),
            jax.ShapeDtypeStruct((_N, _K), jnp.int32),
        ],
        interpret=_INTERP,
    )(x, gate_w, gate_b)


def _route(idx, w):
    idxf = idx.reshape(-1)
    onehot = (idxf[:, None] == jnp.arange(_E)[None, :]).astype(jnp.int32)
    counts = onehot.sum(0)
    pc = ((counts + _B - 1) // _B) * _B
    ends = jnp.cumsum(pc)
    off = ends - pc
    ranks = jnp.cumsum(onehot, 0) - onehot
    r = (ranks * onehot).sum(1)
    pos = off[idxf] + r
    src = jnp.zeros((_ROWS,), jnp.int32).at[pos].set(
        jnp.arange(_P, dtype=jnp.int32) // _K)
    ws = jnp.zeros((_ROWS,), jnp.float32).at[pos].set(w.reshape(-1))
    n_used = jnp.sum(pc) // _B
    te_raw = jnp.searchsorted(ends, jnp.arange(_G, dtype=jnp.int32) * _B,
                              side='right').astype(jnp.int32)
    te = jnp.minimum(te_raw, te_raw[n_used - 1])
    return te, n_used.reshape(1).astype(jnp.int32), src, ws, pos.reshape(_N, _K)


def _gelu(a):
    return a * 0.5 * (1.0 + jax.lax.erf(a * 0.7071067811865476))


def _ffn_kernel(te_ref, nu_ref, src_ref, xb_ref, w1_ref, b1_ref, w2_ref,
                b2_ref, ws_ref, out_ref):
    hs = pl.program_id(0)
    g = pl.program_id(1)

    @pl.when(g < nu_ref[0])
    def _():
        toks = jax.lax.broadcasted_iota(jnp.int32, (_B, _N), 1)
        onehot = jnp.where(src_ref[...] == toks, 1.0, 0.0).astype(jnp.bfloat16)
        xb = jnp.dot(onehot, xb_ref[...], preferred_element_type=jnp.float32)
        a = jnp.dot(xb.astype(jnp.bfloat16), w1_ref[0].astype(jnp.bfloat16),
                    preferred_element_type=jnp.float32) + b1_ref[0]
        y = jnp.dot(_gelu(a).astype(jnp.bfloat16),
                    w2_ref[0].astype(jnp.bfloat16),
                    preferred_element_type=jnp.float32)
        scale = jnp.where(hs == 0, 1.0, 0.0)
        out_ref[...] = (y + scale * b2_ref[0]) * ws_ref[...]


def _ffn(te, n_used, src, xb, W1, b1, W2, b2, ws):
    hh = _H // _HS
    grid_spec = pltpu.PrefetchScalarGridSpec(
        num_scalar_prefetch=2,
        grid=(_HS, _G),
        in_specs=[
            pl.BlockSpec((_B, 1), lambda hs, g, te, nu: (g, 0)),
            pl.BlockSpec((_N, _D), lambda hs, g, te, nu: (0, 0)),
            pl.BlockSpec((1, _D, hh), lambda hs, g, te, nu: (te[g], 0, hs)),
            pl.BlockSpec((1, 1, hh), lambda hs, g, te, nu: (te[g], 0, hs)),
            pl.BlockSpec((1, hh, _D), lambda hs, g, te, nu: (te[g], hs, 0)),
            pl.BlockSpec((1, 1, _D), lambda hs, g, te, nu: (te[g], 0, 0)),
            pl.BlockSpec((_B, 1), lambda hs, g, te, nu: (g, 0)),
        ],
        out_specs=pl.BlockSpec((_B, _D), lambda hs, g, te, nu: (hs * _G + g, 0)),
    )
    return pl.pallas_call(
        _ffn_kernel,
        grid_spec=grid_spec,
        out_shape=jax.ShapeDtypeStruct((_HS * _ROWS, _D), jnp.float32),
        interpret=_INTERP,
    )(te, n_used, src.reshape(_ROWS, 1), xb, W1, b1.reshape(_E, 1, _H), W2,
      b2.reshape(_E, 1, _D), ws.reshape(_ROWS, 1))


_NW = 32
_TPW = _N // _NW
_TCH = 8
_TNB = 2


def _sc_mesh():
    return plsc.VectorSubcoreMesh(core_axis_name="c", subcore_axis_name="s")


def _combine_sc(ys, ps):
    @functools.partial(
        pl.kernel, mesh=_sc_mesh(),
        out_type=jax.ShapeDtypeStruct((_N, _D), jnp.float32),
        scratch_types=[
            pltpu.VMEM((4, _TPW), jnp.int32),
            pltpu.VMEM((_TNB, 4, _TCH, _D), jnp.float32),
        ] + [pltpu.SemaphoreType.DMA] * (5 * _TNB),
    )
    def k(ys_hbm, ps_hbm, out_hbm, ix_v, buf_v, *sems):
        gsems = sems[:4 * _TNB]
        wsems = sems[4 * _TNB:]
        wid = lax.axis_index("s") * 2 + lax.axis_index("c")
        base = wid * _TPW
        for q in range(4):
            pltpu.sync_copy(ps_hbm.at[q, pl.ds(base, _TPW)], ix_v.at[q])
        ncv = _TPW // _TCH
        gh = [None] * ncv
        wh = [None] * ncv

        def gather(c):
            b = c % _TNB
            isl = pl.ds(c * _TCH, _TCH)
            return [
                pltpu.async_copy(ys_hbm.at[ix_v.at[q, isl]],
                                 buf_v.at[b, q], gsems[4 * b + q])
                for q in range(4)
            ]

        def accum_write(c):
            b = c % _TNB

            def body(r, carry):
                for j in range(_D // 16):
                    sl = pl.ds(j * 16, 16)
                    buf_v[b, 0, r, sl] = (
                        (buf_v[b, 0, r, sl] + buf_v[b, 1, r, sl])
                        + (buf_v[b, 2, r, sl] + buf_v[b, 3, r, sl]))
                return carry

            lax.fori_loop(0, _TCH, body, 0)
            return pltpu.async_copy(
                buf_v.at[b, 0], out_hbm.at[pl.ds(base + c * _TCH, _TCH)],
                wsems[b])

        for c in range(ncv):
            if c >= _TNB:
                wh[c - _TNB].wait()
            gh[c] = gather(c)
            if c >= 1:
                for cp in gh[c - 1]:
                    cp.wait()
                wh[c - 1] = accum_write(c - 1)
        for cp in gh[ncv - 1]:
            cp.wait()
        wh[ncv - 1] = accum_write(ncv - 1)
        for c in range(max(0, ncv - _TNB), ncv):
            wh[c].wait()

    return k(ys, ps)


def kernel(x, gate_w, gate_b, W1, b1, W2, b2):
    w, idx = _gate(x, gate_w, gate_b)
    te, n_used, src, ws, pos = _route(idx, w)
    ys = _ffn(te, n_used, src, x.astype(jnp.bfloat16), W1, b1, W2, b2, ws)
    ps = jnp.stack([pos[:, 0], pos[:, 1],
                    pos[:, 0] + _ROWS, pos[:, 1] + _ROWS])
    out = _combine_sc(ys, ps)
    return out


# speedup vs baseline: 1.1970x; 1.0129x over previous
import functools
import jax
import jax.numpy as jnp
from jax import lax
from jax.experimental import pallas as pl
from jax.experimental.pallas import tpu as pltpu
from jax.experimental.pallas import tpu_sc as plsc

_D = 1024
_H = 4096
_E = 8
_K = 2
_N = 2048
_B = 256
_HS = 2
_Q = 2 * _HS
_P = _N * _K
_G = _P // _B + _E
_ROWS = _G * _B

_INTERP = False


def _gate_kernel(x_ref, gw_ref, gb_ref, w_ref, i_ref):
    logits = jnp.dot(x_ref[...], gw_ref[...],
                     preferred_element_type=jnp.float32) + gb_ref[...]
    cols = jax.lax.broadcasted_iota(jnp.int32, logits.shape, 1)
    m1 = jnp.max(logits, axis=1)
    i1 = jnp.argmax(logits, axis=1).astype(jnp.int32)
    masked = jnp.where(cols == i1[:, None], -jnp.inf, logits)
    m2 = jnp.max(masked, axis=1)
    i2 = jnp.argmax(masked, axis=1).astype(jnp.int32)
    e2 = jnp.exp(m2 - m1)
    w1 = 1.0 / (1.0 + e2)
    w2 = e2 / (1.0 + e2)
    w_ref[...] = jnp.stack([w1, w2], axis=1)
    i_ref[...] = jnp.stack([i1, i2], axis=1)


def _gate(x, gate_w, gate_b):
    bt = 256
    return pl.pallas_call(
        _gate_kernel,
        grid=(_N // bt,),
        in_specs=[
            pl.BlockSpec((bt, _D), lambda t: (t, 0)),
            pl.BlockSpec((_D, _E), lambda t: (0, 0)),
            pl.BlockSpec((_E,), lambda t: (0,)),
        ],
        out_specs=[
            pl.BlockSpec((bt, _K), lambda t: (t, 0)),
            pl.BlockSpec((bt, _K), lambda t: (t, 0)),
        ],
        out_shape=[
            jax.ShapeDtypeStruct((_N, _K), jnp.float32),
            jax.ShapeDtypeStruct((_N, _K), jnp.int32),
        ],
        interpret=_INTERP,
    )(x, gate_w, gate_b)


def _route(idx, w):
    idxf = idx.reshape(-1)
    onehot = (idxf[:, None] == jnp.arange(_E)[None, :]).astype(jnp.int32)
    counts = onehot.sum(0)
    pc = ((counts + _B - 1) // _B) * _B
    ends = jnp.cumsum(pc)
    off = ends - pc
    ranks = jnp.cumsum(onehot, 0) - onehot
    r = (ranks * onehot).sum(1)
    pos = off[idxf] + r
    src = jnp.zeros((_ROWS,), jnp.int32).at[pos].set(
        jnp.arange(_P, dtype=jnp.int32) // _K)
    ws = jnp.zeros((_ROWS,), jnp.float32).at[pos].set(w.reshape(-1))
    n_used = jnp.sum(pc) // _B
    te_raw = jnp.searchsorted(ends, jnp.arange(_G, dtype=jnp.int32) * _B,
                              side='right').astype(jnp.int32)
    te = jnp.minimum(te_raw, te_raw[n_used - 1])
    return te, n_used.reshape(1).astype(jnp.int32), src, ws, pos.reshape(_N, _K)


def _gelu(a):
    return a * 0.5 * (1.0 + jax.lax.erf(a * 0.7071067811865476))


def _ffn_kernel(te_ref, nu_ref, src_ref, xb_ref, w1_ref, b1_ref, w2_ref,
                b2_ref, ws_ref, out_ref):
    hs = pl.program_id(0)
    g = pl.program_id(1)

    @pl.when(g < nu_ref[0])
    def _():
        toks = jax.lax.broadcasted_iota(jnp.int32, (_B, _N), 1)
        onehot = jnp.where(src_ref[...] == toks, 1.0, 0.0).astype(jnp.bfloat16)
        xb = jnp.dot(onehot, xb_ref[...], preferred_element_type=jnp.float32)
        a = jnp.dot(xb.astype(jnp.bfloat16), w1_ref[0].astype(jnp.bfloat16),
                    preferred_element_type=jnp.float32) + b1_ref[0]
        y = jnp.dot(_gelu(a).astype(jnp.bfloat16),
                    w2_ref[0].astype(jnp.bfloat16),
                    preferred_element_type=jnp.float32)
        scale = jnp.where(hs == 0, 1.0, 0.0)
        out_ref[...] = (y + scale * b2_ref[0]) * ws_ref[...]


_FFN_VMEM = 63 * 1024 * 1024


def _ffn(te, n_used, src, xb, W1, b1, W2, b2, ws):
    hh = _H // _HS
    grid_spec = pltpu.PrefetchScalarGridSpec(
        num_scalar_prefetch=2,
        grid=(_HS, _G),
        in_specs=[
            pl.BlockSpec((_B, 1), lambda hs, g, te, nu: (g, 0)),
            pl.BlockSpec((_N, _D), lambda hs, g, te, nu: (0, 0)),
            pl.BlockSpec((1, _D, hh), lambda hs, g, te, nu: (te[g], 0, hs)),
            pl.BlockSpec((1, 1, hh), lambda hs, g, te, nu: (te[g], 0, hs)),
            pl.BlockSpec((1, hh, _D), lambda hs, g, te, nu: (te[g], hs, 0)),
            pl.BlockSpec((1, 1, _D), lambda hs, g, te, nu: (te[g], 0, 0)),
            pl.BlockSpec((_B, 1), lambda hs, g, te, nu: (g, 0)),
        ],
        out_specs=pl.BlockSpec((_B, _D), lambda hs, g, te, nu: (hs * _G + g, 0)),
    )
    return pl.pallas_call(
        _ffn_kernel,
        grid_spec=grid_spec,
        out_shape=jax.ShapeDtypeStruct((_HS * _ROWS, _D), jnp.float32),
        compiler_params=pltpu.CompilerParams(vmem_limit_bytes=_FFN_VMEM),
        interpret=_INTERP,
    )(te, n_used, src.reshape(_ROWS, 1), xb, W1, b1.reshape(_E, 1, _H), W2,
      b2.reshape(_E, 1, _D), ws.reshape(_ROWS, 1))


_NW = 32
_TPW = _N // _NW
_TCH = 16 // _HS
_TNB = 2


def _sc_mesh():
    return plsc.VectorSubcoreMesh(core_axis_name="c", subcore_axis_name="s")


def _combine_sc(ys, ps_il):
    ncv = _TPW // _TCH
    rpc = _Q * _TCH

    @functools.partial(
        pl.kernel, mesh=_sc_mesh(),
        out_type=jax.ShapeDtypeStruct((_N, _D), jnp.float32),
        scratch_types=[
            pltpu.VMEM((ncv, rpc), jnp.int32),
            pltpu.VMEM((_TNB, rpc, _D), jnp.float32),
            pltpu.VMEM((_TNB, _TCH, _D), jnp.float32),
        ] + [pltpu.SemaphoreType.DMA] * (2 * _TNB),
    )
    def k(ys_hbm, ps_hbm, out_hbm, ix_v, buf_v, obuf_v, *sems):
        gsems = sems[:_TNB]
        wsems = sems[_TNB:]
        wid = lax.axis_index("s") * 2 + lax.axis_index("c")
        base = wid * _TPW
        for c in range(ncv):
            pltpu.sync_copy(
                ps_hbm.at[pl.ds((base + c * _TCH) * _Q, rpc)], ix_v.at[c])
        gh = [None] * ncv
        wh = [None] * ncv

        def gather(c):
            b = c % _TNB
            return pltpu.async_copy(ys_hbm.at[ix_v.at[c]], buf_v.at[b],
                                    gsems[b])

        def accum_write(c):
            b = c % _TNB

            def body(t, carry):
                for j in range(_D // 16):
                    sl = pl.ds(j * 16, 16)
                    acc = buf_v[b, t * _Q, sl]
                    for q in range(1, _Q):
                        acc = acc + buf_v[b, t * _Q + q, sl]
                    obuf_v[b, t, sl] = acc
                return carry

            lax.fori_loop(0, _TCH, body, 0)
            return pltpu.async_copy(
                obuf_v.at[b], out_hbm.at[pl.ds(base + c * _TCH, _TCH)],
                wsems[b])

        for c in range(ncv):
            gh[c] = gather(c)
            if c >= 1:
                j = c - 1
                if j >= _TNB:
                    wh[j - _TNB].wait()
                gh[j].wait()
                wh[j] = accum_write(j)
        j = ncv - 1
        if j >= _TNB:
            wh[j - _TNB].wait()
        gh[j].wait()
        wh[j] = accum_write(j)
        for c in range(max(0, ncv - _TNB), ncv):
            wh[c].wait()

    return k(ys, ps_il)


def kernel(x, gate_w, gate_b, W1, b1, W2, b2):
    w, idx = _gate(x, gate_w, gate_b)
    te, n_used, src, ws, pos = _route(idx, w)
    ys = _ffn(te, n_used, src, x.astype(jnp.bfloat16), W1, b1, W2, b2, ws)
    parts = [pos[:, 0] + hs * _ROWS for hs in range(_HS)]
    parts += [pos[:, 1] + hs * _ROWS for hs in range(_HS)]
    ps_il = jnp.stack(parts, axis=1).reshape(_N * _Q)
    out = _combine_sc(ys, ps_il)
    return out
